# baseline (speedup 1.0000x reference)
import jax
import jax.numpy as jnp
from jax import lax
from jax.experimental import pallas as pl
from jax.experimental.pallas import tpu as pltpu
from jax.experimental.pallas import tpu_sc as plsc

B, D, C, K = 1024, 256, 128, 10
BB = 512
NEG = -1e30

NW = 32
RPW = B // NW
KC = K // 2


def _tc_body(x_ref, pw_ref, pb_ref, aw_ref, g_ref, q_ref, s0_ref):
    x = x_ref[...]
    logits = jnp.dot(x, pw_ref[...], preferred_element_type=jnp.float32)
    logits = logits + pb_ref[...][None, :]
    m = jnp.max(logits, axis=1, keepdims=True)
    e = jnp.exp(logits - m)
    g_ref[...] = e / jnp.sum(e, axis=1, keepdims=True)

    q = jnp.dot(x, aw_ref[...], preferred_element_type=jnp.float32)
    q_ref[...] = q
    s0_ref[...] = jnp.sum(q * x, axis=1)


_GDN = lax.GatherDimensionNumbers(
    offset_dims=(), collapsed_slice_dims=(0,), start_index_map=(0,))


def _permute(v, idx):
    return lax.gather(v, idx[:, None], dimension_numbers=_GDN,
                      slice_sizes=(1,),
                      mode=lax.GatherScatterMode.PROMISE_IN_BOUNDS)


def _sc_body(retr_hbm, q_hbm, s0_hbm, g_hbm, y_hbm, out_hbm,
             r_v, q_v, s0_v, y_v, o_v, at_v, sem, sem_r0, sem_r1):
    wid = lax.axis_index("s") * 2 + lax.axis_index("c")
    base = wid * RPW
    ab = pl.multiple_of((wid // 4) * 128, 128)
    cb = (wid % 4) * RPW

    sem_r = [sem_r0, sem_r1]
    cp_r = [pltpu.make_async_copy(
        retr_hbm.at[pl.ds(c * KC, KC), pl.ds(base, RPW), :], r_v.at[c],
        sem_r[c]) for c in range(2)]
    cp_r[0].start(); cp_r[1].start()
    cp_q = pltpu.make_async_copy(q_hbm.at[pl.ds(base, RPW)], q_v, sem)
    cp_s = pltpu.make_async_copy(s0_hbm.at[pl.ds(base, RPW)], s0_v, sem)
    cp_y = pltpu.make_async_copy(y_hbm.at[:, pl.ds(ab, 128)], y_v, sem)
    cp_g = pltpu.make_async_copy(g_hbm.at[pl.ds(base, RPW)], o_v, sem)
    cp_q.start(); cp_s.start(); cp_y.start(); cp_g.start()
    cp_q.wait(); cp_s.wait(); cp_y.wait(); cp_g.wait()

    lanes = lax.iota(jnp.int32, 16)
    zeros = jnp.zeros((16,), jnp.int32)
    shuf = [(lanes + (1 << t)) & 15 for t in range(4)]
    neg_pad = jnp.where(lanes > K, jnp.float32(NEG), 0.0)

    def dots(c, i, qv):
        sv = jnp.zeros((16,), jnp.float32)
        for k in range(KC):
            acc = qv[0] * r_v[c, k, i, pl.ds(0, 16)]
            for j in range(1, D // 16):
                acc = acc + qv[j] * r_v[c, k, i, pl.ds(j * 16, 16)]
            for sh in shuf:
                acc = acc + _permute(acc, sh)
            sv = jnp.where(lanes == c * KC + k + 1, acc, sv)
        return sv

    cp_r[0].wait()
    cp_r[1].wait()

    def row(i, _):
        qv = [q_v[i, pl.ds(j * 16, 16)] for j in range(D // 16)]
        s0g = s0_v[pl.ds((i >> 4) << 4, 16)]
        s0b = _permute(s0g, (i & 15) + zeros)
        s = jnp.where(lanes == 0, s0b, neg_pad)
        s = s + dots(0, i, qv) + dots(1, i, qv)
        m = s
        for sh in shuf:
            m = jnp.maximum(m, _permute(m, sh))
        e = jnp.exp(s - m)
        z = e
        for sh in shuf:
            z = z + _permute(z, sh)
        a = e * (jnp.float32(C) / z)
        a0 = _permute(a, zeros)
        for j in range(C // 16):
            sl = pl.ds(j * 16, 16)
            o_v[i, sl] = a0 * o_v[i, sl]
        plsc.store_scatter(at_v, [lanes, i + zeros], a)
        return 0

    lax.fori_loop(0, RPW, row, 0)

    for r in range(RPW // 16):
        rows = r * 16 + lanes
        for k in range(K):
            plsc.addupdate_scatter(
                o_v, [rows, y_v[k, pl.ds(cb + r * 16, 16)]],
                at_v[k + 1, pl.ds(r * 16, 16)])

    cp_o = pltpu.make_async_copy(o_v, out_hbm.at[pl.ds(base, RPW)], sem)
    cp_o.start(); cp_o.wait()


@jax.jit
def _run(graph_embeddings, retrieval_embeddings, pred_W, pred_b, adapter_W,
         retrieval_y):
    g, q, s0 = pl.pallas_call(
        _tc_body,
        grid=(B // BB,),
        in_specs=[
            pl.BlockSpec((BB, D), lambda i: (i, 0)),
            pl.BlockSpec((D, C), lambda i: (0, 0)),
            pl.BlockSpec((C,), lambda i: (0,)),
            pl.BlockSpec((D, D), lambda i: (0, 0)),
        ],
        out_specs=[
            pl.BlockSpec((BB, C), lambda i: (i, 0)),
            pl.BlockSpec((BB, D), lambda i: (i, 0)),
            pl.BlockSpec((BB,), lambda i: (i,)),
        ],
        out_shape=[
            jax.ShapeDtypeStruct((B, C), jnp.float32),
            jax.ShapeDtypeStruct((B, D), jnp.float32),
            jax.ShapeDtypeStruct((B,), jnp.float32),
        ],
        compiler_params=pltpu.CompilerParams(skip_device_barrier=True),
    )(graph_embeddings, pred_W, pred_b, adapter_W)

    mesh = plsc.VectorSubcoreMesh(core_axis_name="c", subcore_axis_name="s")
    fuse = pl.kernel(
        _sc_body,
        out_type=jax.ShapeDtypeStruct((B, C), jnp.float32),
        mesh=mesh,
        compiler_params=pltpu.CompilerParams(needs_layout_passes=False,
                                             skip_device_barrier=True),
        scratch_types=[
            pltpu.VMEM((2, KC, RPW, D), jnp.float32),
            pltpu.VMEM((RPW, D), jnp.float32),
            pltpu.VMEM((RPW,), jnp.float32),
            pltpu.VMEM((K, 128), jnp.int32),
            pltpu.VMEM((RPW, C), jnp.float32),
            pltpu.VMEM((16, RPW), jnp.float32),
            pltpu.SemaphoreType.DMA,
            pltpu.SemaphoreType.DMA,
            pltpu.SemaphoreType.DMA,
        ],
    )
    return fuse(retrieval_embeddings, q, s0, g, retrieval_y)


def kernel(graph_embeddings, retrieval_embeddings, pred_W, pred_b, adapter_W,
           retrieval_y):
    return _run(graph_embeddings, retrieval_embeddings, pred_W, pred_b,
                adapter_W, retrieval_y.astype(jnp.int32))


# speedup vs baseline: 1.3344x; 1.3344x over previous
import jax
import jax.numpy as jnp
from jax import lax
from jax.experimental import pallas as pl
from jax.experimental.pallas import tpu as pltpu
from jax.experimental.pallas import tpu_sc as plsc

B, D, C, K = 1024, 256, 128, 10
BB = 512
NEG = -1e30

NW = 32
RPW = B // NW


def _tc_body(x_ref, retr_ref, pw_ref, pb_ref, aw_ref, gs_ref, att_ref):
    x = x_ref[...]
    logits = jnp.dot(x, pw_ref[...], preferred_element_type=jnp.float32)
    logits = logits + pb_ref[...][None, :]
    m = jnp.max(logits, axis=1, keepdims=True)
    e = jnp.exp(logits - m)
    g = e / jnp.sum(e, axis=1, keepdims=True)

    q = jnp.dot(x, aw_ref[...], preferred_element_type=jnp.float32)
    s0 = jnp.sum(q * x, axis=1)
    sk = jnp.sum(retr_ref[...] * q[None, :, :], axis=2)
    scores = jnp.concatenate([s0[None, :], sk], axis=0)

    sm = jnp.max(scores, axis=0, keepdims=True)
    se = jnp.exp(scores - sm)
    att = se * (jnp.float32(C) / jnp.sum(se, axis=0, keepdims=True))

    gs_ref[...] = att[0][:, None] * g
    att_ref[...] = jnp.concatenate(
        [att[1:], jnp.zeros((16 - K, BB), jnp.float32)], axis=0)


def _sc_body(att_hbm, y_hbm, gs_hbm, out_hbm, a_v, y_v, o_v, sem):
    wid = lax.axis_index("s") * 2 + lax.axis_index("c")
    base = wid * RPW
    ab = pl.multiple_of((wid // 4) * 128, 128)
    cb = (wid % 4) * RPW
    cp_a = pltpu.make_async_copy(att_hbm.at[:, pl.ds(ab, 128)], a_v, sem)
    cp_y = pltpu.make_async_copy(y_hbm.at[:, pl.ds(ab, 128)], y_v, sem)
    cp_g = pltpu.make_async_copy(gs_hbm.at[pl.ds(base, RPW)], o_v, sem)
    cp_a.start(); cp_y.start(); cp_g.start()
    cp_a.wait(); cp_y.wait(); cp_g.wait()

    lanes = lax.iota(jnp.int32, 16)
    for r in range(RPW // 16):
        rows = r * 16 + lanes
        for k in range(K):
            sl = pl.ds(cb + r * 16, 16)
            plsc.addupdate_scatter(o_v, [rows, y_v[k, sl]], a_v[k, sl])

    cp_o = pltpu.make_async_copy(o_v, out_hbm.at[pl.ds(base, RPW)], sem)
    cp_o.start(); cp_o.wait()


@jax.jit
def _run(graph_embeddings, retrieval_embeddings, pred_W, pred_b, adapter_W,
         retrieval_y):
    gs, att = pl.pallas_call(
        _tc_body,
        grid=(B // BB,),
        in_specs=[
            pl.BlockSpec((BB, D), lambda i: (i, 0)),
            pl.BlockSpec((K, BB, D), lambda i: (0, i, 0)),
            pl.BlockSpec((D, C), lambda i: (0, 0)),
            pl.BlockSpec((C,), lambda i: (0,)),
            pl.BlockSpec((D, D), lambda i: (0, 0)),
        ],
        out_specs=[
            pl.BlockSpec((BB, C), lambda i: (i, 0)),
            pl.BlockSpec((16, BB), lambda i: (0, i)),
        ],
        out_shape=[
            jax.ShapeDtypeStruct((B, C), jnp.float32),
            jax.ShapeDtypeStruct((16, B), jnp.float32),
        ],
        compiler_params=pltpu.CompilerParams(skip_device_barrier=True),
    )(graph_embeddings, retrieval_embeddings, pred_W, pred_b, adapter_W)

    mesh = plsc.VectorSubcoreMesh(core_axis_name="c", subcore_axis_name="s")
    fuse = pl.kernel(
        _sc_body,
        out_type=jax.ShapeDtypeStruct((B, C), jnp.float32),
        mesh=mesh,
        compiler_params=pltpu.CompilerParams(needs_layout_passes=False,
                                             skip_device_barrier=True),
        scratch_types=[
            pltpu.VMEM((16, 128), jnp.float32),
            pltpu.VMEM((K, 128), jnp.int32),
            pltpu.VMEM((RPW, C), jnp.float32),
            pltpu.SemaphoreType.DMA,
        ],
    )
    return fuse(att, retrieval_y, gs)


def kernel(graph_embeddings, retrieval_embeddings, pred_W, pred_b, adapter_W,
           retrieval_y):
    return _run(graph_embeddings, retrieval_embeddings, pred_W, pred_b,
                adapter_W, retrieval_y.astype(jnp.int32))


# speedup vs baseline: 1.3364x; 1.0015x over previous
import jax
import jax.numpy as jnp
from jax import lax
from jax.experimental import pallas as pl
from jax.experimental.pallas import tpu as pltpu
from jax.experimental.pallas import tpu_sc as plsc

B, D, C, K = 1024, 256, 128, 10
BB = 512
NEG = -1e30

NW = 32
RPW = B // NW


def _tc_body(x_ref, ra_ref, rb_ref, pw_ref, pb_ref, aw_ref, gs_ref, att_ref):
    x = x_ref[...]
    logits = jnp.dot(x, pw_ref[...], preferred_element_type=jnp.float32)
    logits = logits + pb_ref[...][None, :]
    m = jnp.max(logits, axis=1, keepdims=True)
    e = jnp.exp(logits - m)
    g = e / jnp.sum(e, axis=1, keepdims=True)

    q = jnp.dot(x, aw_ref[...], preferred_element_type=jnp.float32)
    s0 = jnp.sum(q * x, axis=1)
    ska = jnp.sum(ra_ref[...] * q[None, :, :], axis=2)
    skb = jnp.sum(rb_ref[...] * q[None, :, :], axis=2)
    scores = jnp.concatenate([s0[None, :], ska, skb], axis=0)

    sm = jnp.max(scores, axis=0, keepdims=True)
    se = jnp.exp(scores - sm)
    att = se * (jnp.float32(C) / jnp.sum(se, axis=0, keepdims=True))

    gs_ref[...] = att[0][:, None] * g
    att_ref[...] = jnp.concatenate(
        [att[1:], jnp.zeros((16 - K, BB), jnp.float32)], axis=0)


def _sc_body(att_hbm, y_hbm, gs_hbm, out_hbm, a_v, y_v, o_v, sem):
    wid = lax.axis_index("s") * 2 + lax.axis_index("c")
    base = wid * RPW
    ab = pl.multiple_of((wid // 4) * 128, 128)
    cb = (wid % 4) * RPW
    cp_a = pltpu.make_async_copy(att_hbm.at[:, pl.ds(ab, 128)], a_v, sem)
    cp_y = pltpu.make_async_copy(y_hbm.at[:, pl.ds(ab, 128)], y_v, sem)
    cp_g = pltpu.make_async_copy(gs_hbm.at[pl.ds(base, RPW)], o_v, sem)
    cp_a.start(); cp_y.start(); cp_g.start()
    cp_a.wait(); cp_y.wait(); cp_g.wait()

    lanes = lax.iota(jnp.int32, 16)
    for r in range(RPW // 16):
        rows = r * 16 + lanes
        for k in range(K):
            sl = pl.ds(cb + r * 16, 16)
            plsc.addupdate_scatter(o_v, [rows, y_v[k, sl]], a_v[k, sl])

    cp_o = pltpu.make_async_copy(o_v, out_hbm.at[pl.ds(base, RPW)], sem)
    cp_o.start(); cp_o.wait()


@jax.jit
def _run(graph_embeddings, retrieval_embeddings, pred_W, pred_b, adapter_W,
         retrieval_y):
    gs, att = pl.pallas_call(
        _tc_body,
        grid=(B // BB,),
        in_specs=[
            pl.BlockSpec((BB, D), lambda i: (i, 0)),
            pl.BlockSpec((K // 2, BB, D), lambda i: (0, i, 0)),
            pl.BlockSpec((K // 2, BB, D), lambda i: (1, i, 0)),
            pl.BlockSpec((D, C), lambda i: (0, 0)),
            pl.BlockSpec((C,), lambda i: (0,)),
            pl.BlockSpec((D, D), lambda i: (0, 0)),
        ],
        out_specs=[
            pl.BlockSpec((BB, C), lambda i: (i, 0)),
            pl.BlockSpec((16, BB), lambda i: (0, i)),
        ],
        out_shape=[
            jax.ShapeDtypeStruct((B, C), jnp.float32),
            jax.ShapeDtypeStruct((16, B), jnp.float32),
        ],
        compiler_params=pltpu.CompilerParams(skip_device_barrier=True),
    )(graph_embeddings, retrieval_embeddings, retrieval_embeddings,
      pred_W, pred_b, adapter_W)

    mesh = plsc.VectorSubcoreMesh(core_axis_name="c", subcore_axis_name="s")
    fuse = pl.kernel(
        _sc_body,
        out_type=jax.ShapeDtypeStruct((B, C), jnp.float32),
        mesh=mesh,
        compiler_params=pltpu.CompilerParams(needs_layout_passes=False,
                                             skip_device_barrier=True),
        scratch_types=[
            pltpu.VMEM((16, 128), jnp.float32),
            pltpu.VMEM((K, 128), jnp.int32),
            pltpu.VMEM((RPW, C), jnp.float32),
            pltpu.SemaphoreType.DMA,
        ],
    )
    return fuse(att, retrieval_y, gs)


def kernel(graph_embeddings, retrieval_embeddings, pred_W, pred_b, adapter_W,
           retrieval_y):
    return _run(graph_embeddings, retrieval_embeddings, pred_W, pred_b,
                adapter_W, retrieval_y.astype(jnp.int32))


# speedup vs baseline: 1.3991x; 1.0469x over previous
import jax
import jax.numpy as jnp
from jax import lax
from jax.experimental import pallas as pl
from jax.experimental.pallas import tpu as pltpu
from jax.experimental.pallas import tpu_sc as plsc

B, D, C, K = 1024, 256, 128, 10
BB = 512
NEG = -1e30

NW = 16
RPW = B // NW


def _tc_body(x_ref, retr_ref, pw_ref, pb_ref, aw_ref, gs_ref, att_ref):
    x = x_ref[...]
    logits = jnp.dot(x, pw_ref[...], preferred_element_type=jnp.float32)
    logits = logits + pb_ref[...][None, :]
    m = jnp.max(logits, axis=1, keepdims=True)
    e = jnp.exp(logits - m)
    g = e / jnp.sum(e, axis=1, keepdims=True)

    q = jnp.dot(x, aw_ref[...], preferred_element_type=jnp.float32)
    s0 = jnp.sum(q * x, axis=1)
    sk = jnp.sum(retr_ref[...] * q[None, :, :], axis=2)
    scores = jnp.concatenate([s0[None, :], sk], axis=0)

    sm = jnp.max(scores, axis=0, keepdims=True)
    se = jnp.exp(scores - sm)
    att = se * (jnp.float32(C) / jnp.sum(se, axis=0, keepdims=True))

    gs_ref[...] = att[0][:, None] * g
    att_ref[...] = jnp.concatenate(
        [att[1:], jnp.zeros((16 - K, BB), jnp.float32)], axis=0)


def _sc_body(att_hbm, y_hbm, gs_hbm, out_hbm, a_v, y_v, o_v, sem):
    wid = lax.axis_index("s")
    base = wid * RPW
    ab = pl.multiple_of((wid // 2) * 128, 128)
    cb = (wid % 2) * RPW
    cp_a = pltpu.make_async_copy(att_hbm.at[:, pl.ds(ab, 128)], a_v, sem)
    cp_y = pltpu.make_async_copy(y_hbm.at[:, pl.ds(ab, 128)], y_v, sem)
    cp_g = pltpu.make_async_copy(gs_hbm.at[pl.ds(base, RPW)], o_v, sem)
    cp_a.start(); cp_y.start(); cp_g.start()
    cp_a.wait(); cp_y.wait(); cp_g.wait()

    lanes = lax.iota(jnp.int32, 16)
    for r in range(RPW // 16):
        rows = r * 16 + lanes
        for k in range(K):
            sl = pl.ds(cb + r * 16, 16)
            plsc.addupdate_scatter(o_v, [rows, y_v[k, sl]], a_v[k, sl])

    cp_o = pltpu.make_async_copy(o_v, out_hbm.at[pl.ds(base, RPW)], sem)
    cp_o.start(); cp_o.wait()


@jax.jit
def _run(graph_embeddings, retrieval_embeddings, pred_W, pred_b, adapter_W,
         retrieval_y):
    gs, att = pl.pallas_call(
        _tc_body,
        grid=(B // BB,),
        in_specs=[
            pl.BlockSpec((BB, D), lambda i: (i, 0)),
            pl.BlockSpec((K, BB, D), lambda i: (0, i, 0)),
            pl.BlockSpec((D, C), lambda i: (0, 0)),
            pl.BlockSpec((C,), lambda i: (0,)),
            pl.BlockSpec((D, D), lambda i: (0, 0)),
        ],
        out_specs=[
            pl.BlockSpec((BB, C), lambda i: (i, 0)),
            pl.BlockSpec((16, BB), lambda i: (0, i)),
        ],
        out_shape=[
            jax.ShapeDtypeStruct((B, C), jnp.float32),
            jax.ShapeDtypeStruct((16, B), jnp.float32),
        ],
        compiler_params=pltpu.CompilerParams(skip_device_barrier=True),
    )(graph_embeddings, retrieval_embeddings, pred_W, pred_b, adapter_W)

    mesh = plsc.VectorSubcoreMesh(core_axis_name="c", subcore_axis_name="s", num_cores=1)
    fuse = pl.kernel(
        _sc_body,
        out_type=jax.ShapeDtypeStruct((B, C), jnp.float32),
        mesh=mesh,
        compiler_params=pltpu.CompilerParams(needs_layout_passes=False,
                                             skip_device_barrier=True),
        scratch_types=[
            pltpu.VMEM((16, 128), jnp.float32),
            pltpu.VMEM((K, 128), jnp.int32),
            pltpu.VMEM((RPW, C), jnp.float32),
            pltpu.SemaphoreType.DMA,
        ],
    )
    return fuse(att, retrieval_y, gs)


def kernel(graph_embeddings, retrieval_embeddings, pred_W, pred_b, adapter_W,
           retrieval_y):
    return _run(graph_embeddings, retrieval_embeddings, pred_W, pred_b,
                adapter_W, retrieval_y.astype(jnp.int32))
